# baseline (speedup 1.0000x reference)
import functools

import jax
import jax.numpy as jnp
from jax import lax
from jax.experimental import pallas as pl
from jax.experimental.pallas import tpu as pltpu
from jax.experimental.pallas import tpu_sc as plsc

_VOCAB = 1000000
_FEAT = 32
_BATCH = 16384
_HIST = 20
_NW = 32
_VBLK = 128
_NFULL = _VOCAB // _VBLK
_TAIL = _VOCAB - _NFULL * _VBLK
_VPAD = (_NFULL + 1) * _VBLK
_TITER = 246
_BPW = _BATCH // _NW
_SUB = 128
_NSUB = _BPW // _SUB
_NCH = _HIST * _NSUB

_mesh = plsc.VectorSubcoreMesh(core_axis_name="c", subcore_axis_name="s")
_params = pltpu.CompilerParams(use_tc_tiling_on_sc=True,
                               needs_layout_passes=False)


def _wid():
    return lax.axis_index("s") * 2 + lax.axis_index("c")


@functools.partial(
    pl.kernel,
    mesh=_mesh,
    out_type=[
        jax.ShapeDtypeStruct((_VPAD // 4, 128), jnp.float32),
        jax.ShapeDtypeStruct((_VPAD // 4, 128), jnp.float32),
    ],
    scratch_types=[
        [pltpu.VMEM((_FEAT, _VBLK), jnp.float32) for _ in range(2)],
        [pltpu.VMEM((_FEAT, _VBLK), jnp.float32) for _ in range(2)],
        [pltpu.VMEM((_VBLK // 4, 128), jnp.float32) for _ in range(2)],
        [pltpu.VMEM((_VBLK // 4, 128), jnp.float32) for _ in range(2)],
        [pltpu.SemaphoreType.DMA for _ in range(2)],
        [pltpu.SemaphoreType.DMA for _ in range(2)],
        [pltpu.SemaphoreType.DMA for _ in range(2)],
        [pltpu.SemaphoreType.DMA for _ in range(2)],
    ],
    compiler_params=_params,
)
def _transpose_sc(rtT, itT, tail_r, tail_i, sr, si,
                  tin_r, tin_i, stage_r, stage_i,
                  isem_r, isem_i, osem_r, osem_i):
    wid = _wid()
    rows16 = lax.iota(jnp.int32, 16)

    def in_start(blk, buf):
        v0 = blk * _VBLK
        pltpu.async_copy(rtT.at[:, pl.ds(v0, _VBLK)], tin_r[buf], isem_r[buf])
        pltpu.async_copy(itT.at[:, pl.ds(v0, _VBLK)], tin_i[buf], isem_i[buf])

    def in_wait(blk, buf):
        v0 = blk * _VBLK
        pltpu.make_async_copy(rtT.at[:, pl.ds(v0, _VBLK)], tin_r[buf],
                              isem_r[buf]).wait()
        pltpu.make_async_copy(itT.at[:, pl.ds(v0, _VBLK)], tin_i[buf],
                              isem_i[buf]).wait()

    def out_start(blk, buf):
        r0 = blk * (_VBLK // 4)
        pltpu.async_copy(stage_r[buf], sr.at[pl.ds(r0, _VBLK // 4), :],
                         osem_r[buf])
        pltpu.async_copy(stage_i[buf], si.at[pl.ds(r0, _VBLK // 4), :],
                         osem_i[buf])

    def out_wait(blk, buf):
        r0 = blk * (_VBLK // 4)
        pltpu.make_async_copy(stage_r[buf], sr.at[pl.ds(r0, _VBLK // 4), :],
                              osem_r[buf]).wait()
        pltpu.make_async_copy(stage_i[buf], si.at[pl.ds(r0, _VBLK // 4), :],
                              osem_i[buf]).wait()

    def transpose_block(buf):
        def tile_body(i, carry):
            c0 = (i >> 2) * 16
            d0 = (i & 3) * 4
            for tin, stage in ((tin_r, stage_r), (tin_i, stage_i)):
                for r0 in (0, 16):
                    rvec = rows16 + r0
                    for dk in range(4):
                        cvec = c0 + ((rows16 + (d0 + dk)) & 15)
                        v = plsc.load_gather(tin[buf], [rvec, cvec])
                        plsc.store_scatter(
                            stage[buf],
                            [lax.shift_right_logical(cvec, 2),
                             ((cvec & 3) << 5) + rvec], v)
            return carry

        lax.fori_loop(0, (_VBLK // 16) * 4, tile_body, 0)

    in_start(wid, 0)
    in_start(wid + _NW, 1)

    def half(t, buf):
        blk = t * _NW + wid
        prev = blk - 2 * _NW
        nxt = blk + 2 * _NW

        @pl.when(blk < _NFULL)
        def _():
            in_wait(blk, buf)

        @pl.when((prev >= 0) & (prev < _NFULL))
        def _():
            out_wait(prev, buf)

        @pl.when(blk < _NFULL)
        def _():
            transpose_block(buf)
            out_start(blk, buf)

        @pl.when(nxt < _NFULL)
        def _():
            in_start(nxt, buf)

    def pair_body(t2, carry):
        half(t2 * 2, 0)
        half(t2 * 2 + 1, 1)
        return carry

    lax.fori_loop(0, _TITER // 2, pair_body, 0)

    @pl.when(wid + 244 * _NW < _NFULL)
    def _():
        out_wait(wid + 244 * _NW, 0)

    @pl.when(wid == 31)
    def _():
        pltpu.sync_copy(tail_r, sr.at[pl.ds(_NFULL * (_VBLK // 4), _VBLK // 4), :])
        pltpu.sync_copy(tail_i, si.at[pl.ds(_NFULL * (_VBLK // 4), _VBLK // 4), :])


@functools.partial(
    pl.kernel,
    mesh=_mesh,
    out_type=[
        jax.ShapeDtypeStruct((_HIST, _FEAT, _BATCH), jnp.float32),
        jax.ShapeDtypeStruct((_HIST, _FEAT, _BATCH), jnp.float32),
    ],
    scratch_types=[
        [pltpu.VMEM((_SUB,), jnp.int32) for _ in range(2)],
        [pltpu.VMEM((_SUB,), jnp.int32) for _ in range(2)],
        [pltpu.VMEM((_SUB,), jnp.int32) for _ in range(2)],
        [pltpu.VMEM((_SUB, 128), jnp.float32) for _ in range(2)],
        [pltpu.VMEM((_SUB, 128), jnp.float32) for _ in range(2)],
        [pltpu.VMEM((_FEAT, _SUB), jnp.float32) for _ in range(2)],
        [pltpu.VMEM((_FEAT, _SUB), jnp.float32) for _ in range(2)],
        [pltpu.SemaphoreType.DMA for _ in range(2)],
        [pltpu.SemaphoreType.DMA for _ in range(2)],
        [pltpu.SemaphoreType.DMA for _ in range(2)],
        [pltpu.SemaphoreType.DMA for _ in range(2)],
    ],
    compiler_params=_params,
)
def _gather_sc(xT, sr, si, o_r, o_i,
               idx_v, idx4_v, idxm_v, rows_r, rows_i, stage_r, stage_i,
               gsem_r, gsem_i, osem_r, osem_i):
    wid = _wid()
    b0w = wid * _BPW
    rows16 = lax.iota(jnp.int32, 16)

    def pos(k):
        return k // _NSUB, b0w + (k % _NSUB) * _SUB

    def g_start(k, buf):
        h, b0 = pos(k)
        pltpu.sync_copy(xT.at[h, pl.ds(b0, _SUB)], idx_v[buf])
        for g in range(_SUB // 16):
            sl = pl.ds(16 * g, 16)
            iv = idx_v[buf][sl]
            idx4_v[buf][sl] = lax.shift_right_logical(iv, 2)
            idxm_v[buf][sl] = (iv & 3) << 5
        pltpu.async_copy(sr.at[idx4_v[buf]], rows_r[buf], gsem_r[buf])
        pltpu.async_copy(si.at[idx4_v[buf]], rows_i[buf], gsem_i[buf])

    def g_wait(buf):
        pltpu.make_async_copy(sr.at[idx4_v[buf]], rows_r[buf],
                              gsem_r[buf]).wait()
        pltpu.make_async_copy(si.at[idx4_v[buf]], rows_i[buf],
                              gsem_i[buf]).wait()

    def o_start(k, buf):
        h, b0 = pos(k)
        pltpu.async_copy(stage_r[buf], o_r.at[h, :, pl.ds(b0, _SUB)],
                         osem_r[buf])
        pltpu.async_copy(stage_i[buf], o_i.at[h, :, pl.ds(b0, _SUB)],
                         osem_i[buf])

    def o_wait(k, buf):
        h, b0 = pos(k)
        pltpu.make_async_copy(stage_r[buf], o_r.at[h, :, pl.ds(b0, _SUB)],
                              osem_r[buf]).wait()
        pltpu.make_async_copy(stage_i[buf], o_i.at[h, :, pl.ds(b0, _SUB)],
                              osem_i[buf]).wait()

    def transpose_chunk(buf):
        def tile_body(i, carry):
            g = i >> 1
            f0 = (i & 1) * 16
            rvec = rows16 + g * 16
            mvec = idxm_v[buf][pl.ds(16 * g, 16)]
            for rows, stage in ((rows_r, stage_r), (rows_i, stage_i)):
                for d in range(16):
                    fvec = f0 + ((rows16 + d) & 15)
                    v = plsc.load_gather(rows[buf], [rvec, mvec + fvec])
                    plsc.store_scatter(stage[buf], [fvec, rvec], v)
            return carry

        lax.fori_loop(0, (_SUB // 16) * 2, tile_body, 0)

    g_start(0, 0)
    g_start(1, 1)

    def half(k, buf):
        g_wait(buf)

        @pl.when(k >= 2)
        def _():
            o_wait(k - 2, buf)

        transpose_chunk(buf)
        o_start(k, buf)

        @pl.when(k + 2 < _NCH)
        def _():
            g_start(k + 2, buf)

    def pair_body(k2, carry):
        half(k2 * 2, 0)
        half(k2 * 2 + 1, 1)
        return carry

    lax.fori_loop(0, _NCH // 2, pair_body, 0)
    o_wait(_NCH - 2, 0)
    o_wait(_NCH - 1, 1)


def kernel(x, real_table, imag_table):
    xT = x.T.astype(jnp.int32)
    tail_r = jnp.pad(real_table[_NFULL * _VBLK:].reshape(_TAIL // 4, 128),
                     ((0, (_VBLK - _TAIL) // 4), (0, 0)))
    tail_i = jnp.pad(imag_table[_NFULL * _VBLK:].reshape(_TAIL // 4, 128),
                     ((0, (_VBLK - _TAIL) // 4), (0, 0)))
    sr, si = _transpose_sc(real_table.T, imag_table.T, tail_r, tail_i)
    pr, pi = _gather_sc(xT, sr, si)
    return (jnp.transpose(pr, (2, 0, 1)), jnp.transpose(pi, (2, 0, 1)))


# speedup vs baseline: 1.0024x; 1.0024x over previous
import functools

import jax
import jax.numpy as jnp
from jax import lax
from jax.experimental import pallas as pl
from jax.experimental.pallas import tpu as pltpu
from jax.experimental.pallas import tpu_sc as plsc

_VOCAB = 1000000
_FEAT = 32
_BATCH = 16384
_HIST = 20
_NW = 32
_VBLK = 128
_NFULL = _VOCAB // _VBLK
_TAIL = _VOCAB - _NFULL * _VBLK
_VPAD = (_NFULL + 1) * _VBLK
_TITER = 246
_BPW = _BATCH // _NW
_SUB = 128
_NSUB = _BPW // _SUB
_NCH = _HIST * _NSUB

_mesh = plsc.VectorSubcoreMesh(core_axis_name="c", subcore_axis_name="s")
_params = pltpu.CompilerParams(use_tc_tiling_on_sc=True,
                               needs_layout_passes=False)


def _wid():
    return lax.axis_index("s") * 2 + lax.axis_index("c")


@functools.partial(
    pl.kernel,
    mesh=_mesh,
    out_type=[
        jax.ShapeDtypeStruct((_VPAD, 128), jnp.float32),
        jax.ShapeDtypeStruct((_VPAD, 128), jnp.float32),
    ],
    scratch_types=[
        [pltpu.VMEM((_FEAT, _VBLK), jnp.float32) for _ in range(2)],
        [pltpu.VMEM((_FEAT, _VBLK), jnp.float32) for _ in range(2)],
        [pltpu.VMEM((_VBLK, 128), jnp.float32) for _ in range(2)],
        [pltpu.VMEM((_VBLK, 128), jnp.float32) for _ in range(2)],
        [pltpu.SemaphoreType.DMA for _ in range(2)],
        [pltpu.SemaphoreType.DMA for _ in range(2)],
        [pltpu.SemaphoreType.DMA for _ in range(2)],
        [pltpu.SemaphoreType.DMA for _ in range(2)],
    ],
    compiler_params=_params,
)
def _transpose_sc(rtT, itT, tail_r, tail_i, sr, si,
                  tin_r, tin_i, stage_r, stage_i,
                  isem_r, isem_i, osem_r, osem_i):
    wid = _wid()
    rows16 = lax.iota(jnp.int32, 16)

    def in_start(blk, buf):
        v0 = blk * _VBLK
        pltpu.async_copy(rtT.at[:, pl.ds(v0, _VBLK)], tin_r[buf], isem_r[buf])
        pltpu.async_copy(itT.at[:, pl.ds(v0, _VBLK)], tin_i[buf], isem_i[buf])

    def in_wait(blk, buf):
        v0 = blk * _VBLK
        pltpu.make_async_copy(rtT.at[:, pl.ds(v0, _VBLK)], tin_r[buf],
                              isem_r[buf]).wait()
        pltpu.make_async_copy(itT.at[:, pl.ds(v0, _VBLK)], tin_i[buf],
                              isem_i[buf]).wait()

    def out_start(blk, buf):
        v0 = blk * _VBLK
        pltpu.async_copy(stage_r[buf], sr.at[pl.ds(v0, _VBLK), :], osem_r[buf])
        pltpu.async_copy(stage_i[buf], si.at[pl.ds(v0, _VBLK), :], osem_i[buf])

    def out_wait(blk, buf):
        v0 = blk * _VBLK
        pltpu.make_async_copy(stage_r[buf], sr.at[pl.ds(v0, _VBLK), :],
                              osem_r[buf]).wait()
        pltpu.make_async_copy(stage_i[buf], si.at[pl.ds(v0, _VBLK), :],
                              osem_i[buf]).wait()

    def transpose_block(buf):
        def tile_body(i, carry):
            c0 = (i >> 2) * 16
            d0 = (i & 3) * 4
            for tin, stage in ((tin_r, stage_r), (tin_i, stage_i)):
                for r0 in (0, 16):
                    rvec = rows16 + r0
                    for dk in range(4):
                        cvec = c0 + ((rows16 + (d0 + dk)) & 15)
                        v = plsc.load_gather(tin[buf], [rvec, cvec])
                        plsc.store_scatter(stage[buf], [cvec, rvec], v)
            return carry

        lax.fori_loop(0, (_VBLK // 16) * 4, tile_body, 0)

    in_start(wid, 0)
    in_start(wid + _NW, 1)

    def half(t, buf):
        blk = t * _NW + wid
        prev = blk - 2 * _NW
        nxt = blk + 2 * _NW

        @pl.when(blk < _NFULL)
        def _():
            in_wait(blk, buf)

        @pl.when((prev >= 0) & (prev < _NFULL))
        def _():
            out_wait(prev, buf)

        @pl.when(blk < _NFULL)
        def _():
            transpose_block(buf)
            out_start(blk, buf)

        @pl.when(nxt < _NFULL)
        def _():
            in_start(nxt, buf)

    def pair_body(t2, carry):
        half(t2 * 2, 0)
        half(t2 * 2 + 1, 1)
        return carry

    lax.fori_loop(0, _TITER // 2, pair_body, 0)

    @pl.when(wid + 244 * _NW < _NFULL)
    def _():
        out_wait(wid + 244 * _NW, 0)

    @pl.when(wid == 31)
    def _():
        pltpu.sync_copy(tail_r, sr.at[pl.ds(_NFULL * _VBLK, _VBLK), :])
        pltpu.sync_copy(tail_i, si.at[pl.ds(_NFULL * _VBLK, _VBLK), :])


@functools.partial(
    pl.kernel,
    mesh=_mesh,
    out_type=[
        jax.ShapeDtypeStruct((_HIST, _FEAT, _BATCH), jnp.float32),
        jax.ShapeDtypeStruct((_HIST, _FEAT, _BATCH), jnp.float32),
    ],
    scratch_types=[
        [pltpu.VMEM((_SUB,), jnp.int32) for _ in range(2)],
        [pltpu.VMEM((_SUB, 128), jnp.float32) for _ in range(2)],
        [pltpu.VMEM((_SUB, 128), jnp.float32) for _ in range(2)],
        [pltpu.VMEM((_FEAT, _SUB), jnp.float32) for _ in range(2)],
        [pltpu.VMEM((_FEAT, _SUB), jnp.float32) for _ in range(2)],
        [pltpu.SemaphoreType.DMA for _ in range(2)],
        [pltpu.SemaphoreType.DMA for _ in range(2)],
        [pltpu.SemaphoreType.DMA for _ in range(2)],
        [pltpu.SemaphoreType.DMA for _ in range(2)],
    ],
    compiler_params=_params,
)
def _gather_sc(xT, sr, si, o_r, o_i,
               idx_v, rows_r, rows_i, stage_r, stage_i,
               gsem_r, gsem_i, osem_r, osem_i):
    wid = _wid()
    b0w = wid * _BPW
    rows16 = lax.iota(jnp.int32, 16)

    def pos(k):
        return k // _NSUB, b0w + (k % _NSUB) * _SUB

    def g_start(k, buf):
        h, b0 = pos(k)
        pltpu.sync_copy(xT.at[h, pl.ds(b0, _SUB)], idx_v[buf])
        pltpu.async_copy(sr.at[idx_v[buf]], rows_r[buf], gsem_r[buf])
        pltpu.async_copy(si.at[idx_v[buf]], rows_i[buf], gsem_i[buf])

    def g_wait(buf):
        pltpu.make_async_copy(sr.at[idx_v[buf]], rows_r[buf],
                              gsem_r[buf]).wait()
        pltpu.make_async_copy(si.at[idx_v[buf]], rows_i[buf],
                              gsem_i[buf]).wait()

    def o_start(k, buf):
        h, b0 = pos(k)
        pltpu.async_copy(stage_r[buf], o_r.at[h, :, pl.ds(b0, _SUB)],
                         osem_r[buf])
        pltpu.async_copy(stage_i[buf], o_i.at[h, :, pl.ds(b0, _SUB)],
                         osem_i[buf])

    def o_wait(k, buf):
        h, b0 = pos(k)
        pltpu.make_async_copy(stage_r[buf], o_r.at[h, :, pl.ds(b0, _SUB)],
                              osem_r[buf]).wait()
        pltpu.make_async_copy(stage_i[buf], o_i.at[h, :, pl.ds(b0, _SUB)],
                              osem_i[buf]).wait()

    def transpose_chunk(buf):
        def tile_body(i, carry):
            rvec = rows16 + (i >> 3) * 16
            d0 = (i & 7) * 2
            for rows, stage in ((rows_r, stage_r), (rows_i, stage_i)):
                for f0 in (0, 16):
                    for dk in range(2):
                        fvec = f0 + ((rows16 + (d0 + dk)) & 15)
                        v = plsc.load_gather(rows[buf], [rvec, fvec])
                        plsc.store_scatter(stage[buf], [fvec, rvec], v)
            return carry

        lax.fori_loop(0, (_SUB // 16) * 8, tile_body, 0)

    g_start(0, 0)
    g_start(1, 1)

    def half(k, buf):
        g_wait(buf)

        @pl.when(k >= 2)
        def _():
            o_wait(k - 2, buf)

        transpose_chunk(buf)
        o_start(k, buf)

        @pl.when(k + 2 < _NCH)
        def _():
            g_start(k + 2, buf)

    def pair_body(k2, carry):
        half(k2 * 2, 0)
        half(k2 * 2 + 1, 1)
        return carry

    lax.fori_loop(0, _NCH // 2, pair_body, 0)
    o_wait(_NCH - 2, 0)
    o_wait(_NCH - 1, 1)


def kernel(x, real_table, imag_table):
    xT = x.T.astype(jnp.int32)
    tail_r = jnp.pad(real_table[_NFULL * _VBLK:],
                     ((0, _VBLK - _TAIL), (0, 128 - _FEAT)))
    tail_i = jnp.pad(imag_table[_NFULL * _VBLK:],
                     ((0, _VBLK - _TAIL), (0, 128 - _FEAT)))
    sr, si = _transpose_sc(real_table.T, imag_table.T, tail_r, tail_i)
    pr, pi = _gather_sc(xT, sr, si)
    return (jnp.transpose(pr, (2, 0, 1)), jnp.transpose(pi, (2, 0, 1)))


# speedup vs baseline: 1.0132x; 1.0108x over previous
import functools

import jax
import jax.numpy as jnp
from jax import lax
from jax.experimental import pallas as pl
from jax.experimental.pallas import tpu as pltpu
from jax.experimental.pallas import tpu_sc as plsc

_VOCAB = 1000000
_FEAT = 32
_BATCH = 16384
_HIST = 20
_NW = 32
_VBLK = 128
_NFULL = _VOCAB // _VBLK
_TAIL = _VOCAB - _NFULL * _VBLK
_VPAD = (_NFULL + 1) * _VBLK
_TITER = 246
_BPW = _BATCH // _NW
_SUB = 128
_NSUB = _BPW // _SUB
_NCH = _HIST * _NSUB

_mesh = plsc.VectorSubcoreMesh(core_axis_name="c", subcore_axis_name="s")
_params = pltpu.CompilerParams(use_tc_tiling_on_sc=True,
                               needs_layout_passes=False)


def _wid():
    return lax.axis_index("s") * 2 + lax.axis_index("c")


@functools.partial(
    pl.kernel,
    mesh=_mesh,
    out_type=[
        jax.ShapeDtypeStruct((_VPAD, 128), jnp.float32),
        jax.ShapeDtypeStruct((_VPAD, 128), jnp.float32),
    ],
    scratch_types=[
        [pltpu.VMEM((_FEAT, _VBLK), jnp.float32) for _ in range(2)],
        [pltpu.VMEM((_FEAT, _VBLK), jnp.float32) for _ in range(2)],
        [pltpu.VMEM((_VBLK, 128), jnp.float32) for _ in range(2)],
        [pltpu.VMEM((_VBLK, 128), jnp.float32) for _ in range(2)],
        [pltpu.SemaphoreType.DMA for _ in range(2)],
        [pltpu.SemaphoreType.DMA for _ in range(2)],
        [pltpu.SemaphoreType.DMA for _ in range(2)],
        [pltpu.SemaphoreType.DMA for _ in range(2)],
    ],
    compiler_params=_params,
)
def _transpose_sc(rtT, itT, tail_r, tail_i, sr, si,
                  tin_r, tin_i, stage_r, stage_i,
                  isem_r, isem_i, osem_r, osem_i):
    wid = _wid()
    rows16 = lax.iota(jnp.int32, 16)

    def in_start(blk, buf):
        v0 = blk * _VBLK
        pltpu.async_copy(rtT.at[:, pl.ds(v0, _VBLK)], tin_r[buf], isem_r[buf])
        pltpu.async_copy(itT.at[:, pl.ds(v0, _VBLK)], tin_i[buf], isem_i[buf])

    def in_wait(blk, buf):
        v0 = blk * _VBLK
        pltpu.make_async_copy(rtT.at[:, pl.ds(v0, _VBLK)], tin_r[buf],
                              isem_r[buf]).wait()
        pltpu.make_async_copy(itT.at[:, pl.ds(v0, _VBLK)], tin_i[buf],
                              isem_i[buf]).wait()

    def out_start(blk, buf):
        v0 = blk * _VBLK
        pltpu.async_copy(stage_r[buf], sr.at[pl.ds(v0, _VBLK), :], osem_r[buf])
        pltpu.async_copy(stage_i[buf], si.at[pl.ds(v0, _VBLK), :], osem_i[buf])

    def out_wait(blk, buf):
        v0 = blk * _VBLK
        pltpu.make_async_copy(stage_r[buf], sr.at[pl.ds(v0, _VBLK), :],
                              osem_r[buf]).wait()
        pltpu.make_async_copy(stage_i[buf], si.at[pl.ds(v0, _VBLK), :],
                              osem_i[buf]).wait()

    def transpose_block(buf):
        def tile_body(i, carry):
            c0 = (i >> 1) * 16
            d0 = (i & 1) * 8
            for tin, stage in ((tin_r, stage_r), (tin_i, stage_i)):
                for r0 in (0, 16):
                    rvec = rows16 + r0
                    for dk in range(8):
                        cvec = c0 + ((rows16 + (d0 + dk)) & 15)
                        v = plsc.load_gather(tin[buf], [rvec, cvec])
                        plsc.store_scatter(stage[buf], [cvec, rvec], v)
            return carry

        lax.fori_loop(0, (_VBLK // 16) * 2, tile_body, 0)

    in_start(wid, 0)
    in_start(wid + _NW, 1)

    def half(t, buf):
        blk = t * _NW + wid
        prev = blk - 2 * _NW
        nxt = blk + 2 * _NW

        @pl.when(blk < _NFULL)
        def _():
            in_wait(blk, buf)

        @pl.when((prev >= 0) & (prev < _NFULL))
        def _():
            out_wait(prev, buf)

        @pl.when(blk < _NFULL)
        def _():
            transpose_block(buf)
            out_start(blk, buf)

        @pl.when(nxt < _NFULL)
        def _():
            in_start(nxt, buf)

    def pair_body(t2, carry):
        half(t2 * 2, 0)
        half(t2 * 2 + 1, 1)
        return carry

    lax.fori_loop(0, _TITER // 2, pair_body, 0)

    @pl.when(wid + 244 * _NW < _NFULL)
    def _():
        out_wait(wid + 244 * _NW, 0)

    @pl.when(wid == 31)
    def _():
        pltpu.sync_copy(tail_r, sr.at[pl.ds(_NFULL * _VBLK, _VBLK), :])
        pltpu.sync_copy(tail_i, si.at[pl.ds(_NFULL * _VBLK, _VBLK), :])


@functools.partial(
    pl.kernel,
    mesh=_mesh,
    out_type=[
        jax.ShapeDtypeStruct((_HIST, _FEAT, _BATCH), jnp.float32),
        jax.ShapeDtypeStruct((_HIST, _FEAT, _BATCH), jnp.float32),
    ],
    scratch_types=[
        [pltpu.VMEM((_SUB,), jnp.int32) for _ in range(2)],
        [pltpu.VMEM((_SUB, 128), jnp.float32) for _ in range(2)],
        [pltpu.VMEM((_SUB, 128), jnp.float32) for _ in range(2)],
        [pltpu.VMEM((_FEAT, _SUB), jnp.float32) for _ in range(2)],
        [pltpu.VMEM((_FEAT, _SUB), jnp.float32) for _ in range(2)],
        [pltpu.SemaphoreType.DMA for _ in range(2)],
        [pltpu.SemaphoreType.DMA for _ in range(2)],
        [pltpu.SemaphoreType.DMA for _ in range(2)],
        [pltpu.SemaphoreType.DMA for _ in range(2)],
    ],
    compiler_params=_params,
)
def _gather_sc(xT, sr, si, o_r, o_i,
               idx_v, rows_r, rows_i, stage_r, stage_i,
               gsem_r, gsem_i, osem_r, osem_i):
    wid = _wid()
    b0w = wid * _BPW
    rows16 = lax.iota(jnp.int32, 16)

    def pos(k):
        return k // _NSUB, b0w + (k % _NSUB) * _SUB

    def g_start(k, buf):
        h, b0 = pos(k)
        pltpu.sync_copy(xT.at[h, pl.ds(b0, _SUB)], idx_v[buf])
        pltpu.async_copy(sr.at[idx_v[buf]], rows_r[buf], gsem_r[buf])
        pltpu.async_copy(si.at[idx_v[buf]], rows_i[buf], gsem_i[buf])

    def g_wait(buf):
        pltpu.make_async_copy(sr.at[idx_v[buf]], rows_r[buf],
                              gsem_r[buf]).wait()
        pltpu.make_async_copy(si.at[idx_v[buf]], rows_i[buf],
                              gsem_i[buf]).wait()

    def o_start(k, buf):
        h, b0 = pos(k)
        pltpu.async_copy(stage_r[buf], o_r.at[h, :, pl.ds(b0, _SUB)],
                         osem_r[buf])
        pltpu.async_copy(stage_i[buf], o_i.at[h, :, pl.ds(b0, _SUB)],
                         osem_i[buf])

    def o_wait(k, buf):
        h, b0 = pos(k)
        pltpu.make_async_copy(stage_r[buf], o_r.at[h, :, pl.ds(b0, _SUB)],
                              osem_r[buf]).wait()
        pltpu.make_async_copy(stage_i[buf], o_i.at[h, :, pl.ds(b0, _SUB)],
                              osem_i[buf]).wait()

    def transpose_chunk(buf):
        def tile_body(i, carry):
            rvec = rows16 + (i >> 3) * 16
            d0 = (i & 7) * 2
            for rows, stage in ((rows_r, stage_r), (rows_i, stage_i)):
                for f0 in (0, 16):
                    for dk in range(2):
                        fvec = f0 + ((rows16 + (d0 + dk)) & 15)
                        v = plsc.load_gather(rows[buf], [rvec, fvec])
                        plsc.store_scatter(stage[buf], [fvec, rvec], v)
            return carry

        lax.fori_loop(0, (_SUB // 16) * 8, tile_body, 0)

    g_start(0, 0)
    g_start(1, 1)

    def half(k, buf):
        g_wait(buf)

        @pl.when(k >= 2)
        def _():
            o_wait(k - 2, buf)

        transpose_chunk(buf)
        o_start(k, buf)

        @pl.when(k + 2 < _NCH)
        def _():
            g_start(k + 2, buf)

    def pair_body(k2, carry):
        half(k2 * 2, 0)
        half(k2 * 2 + 1, 1)
        return carry

    lax.fori_loop(0, _NCH // 2, pair_body, 0)
    o_wait(_NCH - 2, 0)
    o_wait(_NCH - 1, 1)


def kernel(x, real_table, imag_table):
    xT = x.T.astype(jnp.int32)
    tail_r = jnp.pad(real_table[_NFULL * _VBLK:],
                     ((0, _VBLK - _TAIL), (0, 128 - _FEAT)))
    tail_i = jnp.pad(imag_table[_NFULL * _VBLK:],
                     ((0, _VBLK - _TAIL), (0, 128 - _FEAT)))
    sr, si = _transpose_sc(real_table.T, imag_table.T, tail_r, tail_i)
    pr, pi = _gather_sc(xT, sr, si)
    return (jnp.transpose(pr, (2, 0, 1)), jnp.transpose(pi, (2, 0, 1)))


# speedup vs baseline: 1.0207x; 1.0073x over previous
import functools

import jax
import jax.numpy as jnp
from jax import lax
from jax.experimental import pallas as pl
from jax.experimental.pallas import tpu as pltpu
from jax.experimental.pallas import tpu_sc as plsc

_VOCAB = 1000000
_FEAT = 32
_BATCH = 16384
_HIST = 20
_NW = 32
_VBLK = 128
_NFULL = _VOCAB // _VBLK
_TAIL = _VOCAB - _NFULL * _VBLK
_VPAD = (_NFULL + 1) * _VBLK
_TITER = 246
_BPW = _BATCH // _NW
_SUB = 128
_NSUB = _BPW // _SUB
_NCH = _HIST * _NSUB

_mesh = plsc.VectorSubcoreMesh(core_axis_name="c", subcore_axis_name="s")
_params = pltpu.CompilerParams(use_tc_tiling_on_sc=True,
                               needs_layout_passes=False)


def _wid():
    return lax.axis_index("s") * 2 + lax.axis_index("c")


@functools.partial(
    pl.kernel,
    mesh=_mesh,
    out_type=[
        jax.ShapeDtypeStruct((_VPAD, 128), jnp.float32),
        jax.ShapeDtypeStruct((_VPAD, 128), jnp.float32),
    ],
    scratch_types=[
        [pltpu.VMEM((_FEAT, _VBLK), jnp.float32) for _ in range(2)],
        [pltpu.VMEM((_FEAT, _VBLK), jnp.float32) for _ in range(2)],
        [pltpu.VMEM((_VBLK, 128), jnp.float32) for _ in range(2)],
        [pltpu.VMEM((_VBLK, 128), jnp.float32) for _ in range(2)],
        [pltpu.SemaphoreType.DMA for _ in range(2)],
        [pltpu.SemaphoreType.DMA for _ in range(2)],
        [pltpu.SemaphoreType.DMA for _ in range(2)],
        [pltpu.SemaphoreType.DMA for _ in range(2)],
    ],
    compiler_params=_params,
)
def _transpose_sc(rtT, itT, tail_r, tail_i, sr, si,
                  tin_r, tin_i, stage_r, stage_i,
                  isem_r, isem_i, osem_r, osem_i):
    wid = _wid()
    rows16 = lax.iota(jnp.int32, 16)

    def in_start(blk, buf):
        v0 = blk * _VBLK
        pltpu.async_copy(rtT.at[:, pl.ds(v0, _VBLK)], tin_r[buf], isem_r[buf])
        pltpu.async_copy(itT.at[:, pl.ds(v0, _VBLK)], tin_i[buf], isem_i[buf])

    def in_wait(blk, buf):
        v0 = blk * _VBLK
        pltpu.make_async_copy(rtT.at[:, pl.ds(v0, _VBLK)], tin_r[buf],
                              isem_r[buf]).wait()
        pltpu.make_async_copy(itT.at[:, pl.ds(v0, _VBLK)], tin_i[buf],
                              isem_i[buf]).wait()

    def out_start(blk, buf):
        v0 = blk * _VBLK
        pltpu.async_copy(stage_r[buf], sr.at[pl.ds(v0, _VBLK), :], osem_r[buf])
        pltpu.async_copy(stage_i[buf], si.at[pl.ds(v0, _VBLK), :], osem_i[buf])

    def out_wait(blk, buf):
        v0 = blk * _VBLK
        pltpu.make_async_copy(stage_r[buf], sr.at[pl.ds(v0, _VBLK), :],
                              osem_r[buf]).wait()
        pltpu.make_async_copy(stage_i[buf], si.at[pl.ds(v0, _VBLK), :],
                              osem_i[buf]).wait()

    def transpose_block(buf):
        def tile_body(i, carry):
            c0 = (i >> 1) * 16
            d0 = (i & 1) * 8
            for tin, stage in ((tin_r, stage_r), (tin_i, stage_i)):
                for r0 in (0, 16):
                    rvec = rows16 + r0
                    for dk in range(8):
                        cvec = c0 + ((rows16 + (d0 + dk)) & 15)
                        v = plsc.load_gather(tin[buf], [rvec, cvec])
                        plsc.store_scatter(stage[buf], [cvec, rvec], v)
            return carry

        lax.fori_loop(0, (_VBLK // 16) * 2, tile_body, 0)

    in_start(wid, 0)
    in_start(wid + _NW, 1)

    def half(t, buf):
        blk = t * _NW + wid
        prev = blk - 2 * _NW
        nxt = blk + 2 * _NW

        @pl.when(blk < _NFULL)
        def _():
            in_wait(blk, buf)

        @pl.when((prev >= 0) & (prev < _NFULL))
        def _():
            out_wait(prev, buf)

        @pl.when(blk < _NFULL)
        def _():
            transpose_block(buf)
            out_start(blk, buf)

        @pl.when(nxt < _NFULL)
        def _():
            in_start(nxt, buf)

    def pair_body(t2, carry):
        half(t2 * 2, 0)
        half(t2 * 2 + 1, 1)
        return carry

    lax.fori_loop(0, _TITER // 2, pair_body, 0)

    @pl.when(wid + 244 * _NW < _NFULL)
    def _():
        out_wait(wid + 244 * _NW, 0)

    @pl.when(wid == 31)
    def _():
        pltpu.sync_copy(tail_r, sr.at[pl.ds(_NFULL * _VBLK, _VBLK), :])
        pltpu.sync_copy(tail_i, si.at[pl.ds(_NFULL * _VBLK, _VBLK), :])


@functools.partial(
    pl.kernel,
    mesh=_mesh,
    out_type=[
        jax.ShapeDtypeStruct((_HIST, _FEAT, _BATCH), jnp.float32),
        jax.ShapeDtypeStruct((_HIST, _FEAT, _BATCH), jnp.float32),
    ],
    scratch_types=[
        [pltpu.VMEM((_SUB,), jnp.int32) for _ in range(2)],
        [pltpu.VMEM((_SUB, 128), jnp.float32) for _ in range(2)],
        [pltpu.VMEM((_SUB, 128), jnp.float32) for _ in range(2)],
        [pltpu.VMEM((_FEAT, _SUB), jnp.float32) for _ in range(2)],
        [pltpu.VMEM((_FEAT, _SUB), jnp.float32) for _ in range(2)],
        [pltpu.SemaphoreType.DMA for _ in range(2)],
        [pltpu.SemaphoreType.DMA for _ in range(2)],
        [pltpu.SemaphoreType.DMA for _ in range(2)],
        [pltpu.SemaphoreType.DMA for _ in range(2)],
    ],
    compiler_params=_params,
)
def _gather_sc(xT, sr, si, o_r, o_i,
               idx_v, rows_r, rows_i, stage_r, stage_i,
               gsem_r, gsem_i, osem_r, osem_i):
    wid = _wid()
    b0w = wid * _BPW
    rows16 = lax.iota(jnp.int32, 16)

    def pos(k):
        return k // _NSUB, b0w + (k % _NSUB) * _SUB

    def g_start(k, buf):
        h, b0 = pos(k)
        pltpu.sync_copy(xT.at[h, pl.ds(b0, _SUB)], idx_v[buf])
        pltpu.async_copy(sr.at[idx_v[buf]], rows_r[buf], gsem_r[buf])
        pltpu.async_copy(si.at[idx_v[buf]], rows_i[buf], gsem_i[buf])

    def g_wait(buf):
        pltpu.make_async_copy(sr.at[idx_v[buf]], rows_r[buf],
                              gsem_r[buf]).wait()
        pltpu.make_async_copy(si.at[idx_v[buf]], rows_i[buf],
                              gsem_i[buf]).wait()

    def o_start(k, buf):
        h, b0 = pos(k)
        pltpu.async_copy(stage_r[buf], o_r.at[h, :, pl.ds(b0, _SUB)],
                         osem_r[buf])
        pltpu.async_copy(stage_i[buf], o_i.at[h, :, pl.ds(b0, _SUB)],
                         osem_i[buf])

    def o_wait(k, buf):
        h, b0 = pos(k)
        pltpu.make_async_copy(stage_r[buf], o_r.at[h, :, pl.ds(b0, _SUB)],
                              osem_r[buf]).wait()
        pltpu.make_async_copy(stage_i[buf], o_i.at[h, :, pl.ds(b0, _SUB)],
                              osem_i[buf]).wait()

    def transpose_chunk(buf):
        def tile_body(i, carry):
            rvec = rows16 + (i >> 1) * 16
            d0 = (i & 1) * 8
            for rows, stage in ((rows_r, stage_r), (rows_i, stage_i)):
                for f0 in (0, 16):
                    for dk in range(8):
                        fvec = f0 + ((rows16 + (d0 + dk)) & 15)
                        v = plsc.load_gather(rows[buf], [rvec, fvec])
                        plsc.store_scatter(stage[buf], [fvec, rvec], v)
            return carry

        lax.fori_loop(0, (_SUB // 16) * 2, tile_body, 0)

    g_start(0, 0)
    g_start(1, 1)

    def half(k, buf):
        g_wait(buf)

        @pl.when(k >= 2)
        def _():
            o_wait(k - 2, buf)

        transpose_chunk(buf)
        o_start(k, buf)

        @pl.when(k + 2 < _NCH)
        def _():
            g_start(k + 2, buf)

    def pair_body(k2, carry):
        half(k2 * 2, 0)
        half(k2 * 2 + 1, 1)
        return carry

    lax.fori_loop(0, _NCH // 2, pair_body, 0)
    o_wait(_NCH - 2, 0)
    o_wait(_NCH - 1, 1)


def kernel(x, real_table, imag_table):
    xT = x.T.astype(jnp.int32)
    tail_r = jnp.pad(real_table[_NFULL * _VBLK:],
                     ((0, _VBLK - _TAIL), (0, 128 - _FEAT)))
    tail_i = jnp.pad(imag_table[_NFULL * _VBLK:],
                     ((0, _VBLK - _TAIL), (0, 128 - _FEAT)))
    sr, si = _transpose_sc(real_table.T, imag_table.T, tail_r, tail_i)
    pr, pi = _gather_sc(xT, sr, si)
    return (jnp.transpose(pr, (2, 0, 1)), jnp.transpose(pi, (2, 0, 1)))
